# baseline (speedup 1.0000x reference)
import functools

import jax
import jax.numpy as jnp
from jax import lax
from jax.experimental import pallas as pl
from jax.experimental.pallas import tpu as pltpu
from jax.experimental.pallas import tpu_sc as plsc

D_MODEL = 1024
D_FF = 2048
N_EXP = 8
N_TOK = 4096
N_PAIR = 2 * N_TOK
TD = 256
NB = N_PAIR // TD + N_EXP
NPAD = NB * TD
NW = 32
TPW = N_TOK // NW
CHD = 32
CHC = 16
WGL = 128

_HI = jax.lax.Precision.HIGHEST


def _router_body(x_ref, wr_ref, i1_ref, i2_ref, wa_ref, wb_ref):
    xb = x_ref[...]
    logits = lax.dot_general(xb, wr_ref[...], (((1,), (1,)), ((), ())),
                             preferred_element_type=jnp.float32)
    p = jax.nn.softmax(logits, axis=-1)
    i1 = jnp.argmax(p, axis=-1)
    v1 = jnp.max(p, axis=-1)
    lane = lax.broadcasted_iota(jnp.int32, p.shape, 1)
    pm = jnp.where(lane == i1[:, None], -1.0, p)
    i2 = jnp.argmax(pm, axis=-1)
    v2 = jnp.max(pm, axis=-1)
    denom = v1 + v2
    i1_ref[...] = i1.astype(jnp.int32)
    i2_ref[...] = i2.astype(jnp.int32)
    wa_ref[...] = v1 / denom
    wb_ref[...] = v2 / denom


def _book_body(pe_ref, pos_ref, beo_ref):
    pe = pe_ref[...]
    r128 = lax.broadcasted_iota(jnp.int32, (128, 128), 0)
    c128 = lax.broadcasted_iota(jnp.int32, (128, 128), 1)
    U = (r128 <= c128).astype(jnp.float32)
    r64 = lax.broadcasted_iota(jnp.int32, (64, 64), 0)
    c64 = lax.broadcasted_iota(jnp.int32, (64, 64), 1)
    L = (c64 < r64).astype(jnp.float32)

    pos_acc = jnp.zeros((64, 128), jnp.float32)
    be_acc = jnp.zeros((8, 128), jnp.int32)
    bidx = lax.broadcasted_iota(jnp.int32, (8, 128), 1)
    offs = jnp.int32(0)
    for e in range(N_EXP):
        m = (pe == e).astype(jnp.float32)
        cs = lax.dot_general(m, U, (((1,), (0,)), ((), ())),
                             precision=_HI, preferred_element_type=jnp.float32)
        rt = cs[:, 127:128]
        O = lax.dot_general(L, rt, (((1,), (0,)), ((), ())),
                            precision=_HI, preferred_element_type=jnp.float32)
        cnt = jnp.sum(rt).astype(jnp.int32)
        offs_f = offs.astype(jnp.float32)
        pos_acc = pos_acc + m * (offs_f + O + cs - 1.0)
        be_acc = be_acc + (bidx >= offs // TD).astype(jnp.int32)
        pc = ((cnt + TD - 1) // TD) * TD
        offs = offs + pc
    pos_ref[...] = pos_acc.astype(jnp.int32)
    beo_ref[...] = be_acc - 1


def _dispatch_body(x_hbm, pose_hbm, poso_hbm, w0_hbm, w1_hbm,
                   xg_hbm, wg_hbm,
                   pose_v, poso_v, w0_v, w1_v,
                   rows_a, rows_b, wg0a, wg0b, wg1a, wg1b, semr, semw):
    wid = lax.axis_index("s") * 2 + lax.axis_index("c")
    pltpu.sync_copy(pose_hbm.at[wid], pose_v)
    pltpu.sync_copy(poso_hbm.at[wid], poso_v)
    pltpu.sync_copy(w0_hbm.at[pl.ds(wid * TPW, TPW)], w0_v)
    pltpu.sync_copy(w1_hbm.at[pl.ds(wid * TPW, TPW)], w1_v)
    nch = TPW // CHD
    rows = (rows_a, rows_b)
    wg0 = (wg0a, wg0b)
    wg1 = (wg1a, wg1b)
    rd = [None] * nch
    wr = [None] * nch
    rd[0] = pltpu.async_copy(x_hbm.at[pl.ds(wid * TPW, CHD)], rows_a, semr)
    for c in range(nch):
        if c >= 1:
            for d in wr[c - 1]:
                d.wait()
        if c + 1 < nch:
            rd[c + 1] = pltpu.async_copy(
                x_hbm.at[pl.ds(wid * TPW + (c + 1) * CHD, CHD)],
                rows[(c + 1) % 2], semr)
        b = c % 2
        for i in range(CHD):
            v0 = w0_v[pl.ds(c * CHD + (i // 16) * 16, 16)]
            v1 = w1_v[pl.ds(c * CHD + (i // 16) * 16, 16)]
            f0 = jnp.full((16,), v0[i % 16], jnp.float32)
            f1 = jnp.full((16,), v1[i % 16], jnp.float32)
            for j in range(WGL // 16):
                wg0[b][i, pl.ds(j * 16, 16)] = f0
                wg1[b][i, pl.ds(j * 16, 16)] = f1
        rd[c].wait()
        wr[c] = [
            pltpu.async_copy(rows[b], xg_hbm.at[pose_v.at[c]], semw),
            pltpu.async_copy(rows[b], xg_hbm.at[poso_v.at[c]], semw),
            pltpu.async_copy(wg0[b], wg_hbm.at[pose_v.at[c]], semw),
            pltpu.async_copy(wg1[b], wg_hbm.at[poso_v.at[c]], semw),
        ]
    for d in wr[nch - 1]:
        d.wait()


def _ffn_body(be_ref, xg_ref, w1_ref, w2_ref, wg_ref, yg_ref):
    xb = xg_ref[...]
    h = lax.dot_general(xb, w1_ref[0], (((1,), (1,)), ((), ())),
                        preferred_element_type=jnp.float32)
    h = (h * jax.nn.sigmoid(h)).astype(jnp.bfloat16)
    y = lax.dot_general(h, w2_ref[0], (((1,), (1,)), ((), ())),
                        preferred_element_type=jnp.float32)
    yg_ref[...] = y * wg_ref[...][:, 0:1]


def _combine_body(yg_hbm, pose_hbm, poso_hbm, out_hbm,
                  pose_v, poso_v, ye_a, ye_b, yo_a, yo_b, o_a, o_b, semr, semw):
    wid = lax.axis_index("s") * 2 + lax.axis_index("c")
    pltpu.sync_copy(pose_hbm.at[wid], pose_v)
    pltpu.sync_copy(poso_hbm.at[wid], poso_v)
    nch = TPW // CHC
    ye = (ye_a, ye_b)
    yo = (yo_a, yo_b)
    ov = (o_a, o_b)
    gd = [None] * nch
    ww = [None] * nch
    gd[0] = [pltpu.async_copy(yg_hbm.at[pose_v.at[0]], ye_a, semr),
             pltpu.async_copy(yg_hbm.at[poso_v.at[0]], yo_a, semr)]
    for c in range(nch):
        b = c % 2
        if c + 1 < nch:
            nb = (c + 1) % 2
            gd[c + 1] = [
                pltpu.async_copy(yg_hbm.at[pose_v.at[c + 1]], ye[nb], semr),
                pltpu.async_copy(yg_hbm.at[poso_v.at[c + 1]], yo[nb], semr)]
        for d in gd[c]:
            d.wait()
        if c >= 2:
            ww[c - 2].wait()

        def row(i, _, b=b):
            for j in range(D_MODEL // 16):
                sl = pl.ds(j * 16, 16)
                ov[b][i, sl] = ye[b][i, sl] + yo[b][i, sl]
            return 0

        lax.fori_loop(0, CHC, row, 0)
        ww[c] = pltpu.async_copy(
            ov[b], out_hbm.at[pl.ds(wid * TPW + c * CHC, CHC)], semw)
    ww[nch - 2].wait()
    ww[nch - 1].wait()


@jax.jit
def kernel(x, Wr, W1, W2):
    b, s, d = x.shape
    x2 = x.reshape(N_TOK, d)
    w1b = W1.astype(jnp.bfloat16)
    w2b = W2.astype(jnp.bfloat16)

    i1, i2, wa, wb = pl.pallas_call(
        _router_body,
        out_shape=[
            jax.ShapeDtypeStruct((N_TOK,), jnp.int32),
            jax.ShapeDtypeStruct((N_TOK,), jnp.int32),
            jax.ShapeDtypeStruct((N_TOK,), jnp.float32),
            jax.ShapeDtypeStruct((N_TOK,), jnp.float32),
        ],
    )(x2, Wr)

    pair_e = jnp.stack([i1, i2], axis=1).reshape(64, 128)
    pos, beo = pl.pallas_call(
        _book_body,
        out_shape=[
            jax.ShapeDtypeStruct((64, 128), jnp.int32),
            jax.ShapeDtypeStruct((8, 128), jnp.int32),
        ],
    )(pair_e)

    pos2 = pos.reshape(N_TOK, 2)
    pose_d = pos2[:, 0].reshape(NW, TPW // CHD, CHD)
    poso_d = pos2[:, 1].reshape(NW, TPW // CHD, CHD)
    pose_c = pos2[:, 0].reshape(NW, TPW // CHC, CHC)
    poso_c = pos2[:, 1].reshape(NW, TPW // CHC, CHC)
    be = beo[0, :NB]

    mesh = plsc.VectorSubcoreMesh(core_axis_name="c", subcore_axis_name="s")
    x2i = lax.bitcast_convert_type(
        x2.astype(jnp.bfloat16).reshape(N_TOK, D_MODEL // 2, 2), jnp.int32)
    dispatch = functools.partial(
        pl.kernel,
        mesh=mesh,
        out_type=[
            jax.ShapeDtypeStruct((NPAD, D_MODEL // 2), jnp.int32),
            jax.ShapeDtypeStruct((NPAD, WGL), jnp.float32),
        ],
        scratch_types=[
            pltpu.VMEM((TPW // CHD, CHD), jnp.int32),
            pltpu.VMEM((TPW // CHD, CHD), jnp.int32),
            pltpu.VMEM((TPW,), jnp.float32),
            pltpu.VMEM((TPW,), jnp.float32),
            pltpu.VMEM((CHD, D_MODEL // 2), jnp.int32),
            pltpu.VMEM((CHD, D_MODEL // 2), jnp.int32),
            pltpu.VMEM((CHD, WGL), jnp.float32),
            pltpu.VMEM((CHD, WGL), jnp.float32),
            pltpu.VMEM((CHD, WGL), jnp.float32),
            pltpu.VMEM((CHD, WGL), jnp.float32),
            pltpu.SemaphoreType.DMA,
            pltpu.SemaphoreType.DMA,
        ],
    )(_dispatch_body)
    xgi, wg = dispatch(x2i, pose_d, poso_d, wa, wb)
    xg = lax.bitcast_convert_type(xgi, jnp.bfloat16).reshape(NPAD, D_MODEL)

    grid_spec = pltpu.PrefetchScalarGridSpec(
        num_scalar_prefetch=1,
        grid=(NB,),
        in_specs=[
            pl.BlockSpec((TD, D_MODEL), lambda bb, be_r: (bb, 0)),
            pl.BlockSpec((1, D_FF, D_MODEL), lambda bb, be_r: (be_r[bb], 0, 0)),
            pl.BlockSpec((1, D_MODEL, D_FF), lambda bb, be_r: (be_r[bb], 0, 0)),
            pl.BlockSpec((TD, WGL), lambda bb, be_r: (bb, 0)),
        ],
        out_specs=pl.BlockSpec((TD, D_MODEL), lambda bb, be_r: (bb, 0)),
    )
    yg = pl.pallas_call(
        _ffn_body,
        grid_spec=grid_spec,
        out_shape=jax.ShapeDtypeStruct((NPAD, D_MODEL), jnp.float32),
    )(be, xg, w1b, w2b, wg)

    combine = functools.partial(
        pl.kernel,
        mesh=mesh,
        out_type=jax.ShapeDtypeStruct((N_TOK, D_MODEL), jnp.float32),
        scratch_types=[
            pltpu.VMEM((TPW // CHC, CHC), jnp.int32),
            pltpu.VMEM((TPW // CHC, CHC), jnp.int32),
            pltpu.VMEM((CHC, D_MODEL), jnp.float32),
            pltpu.VMEM((CHC, D_MODEL), jnp.float32),
            pltpu.VMEM((CHC, D_MODEL), jnp.float32),
            pltpu.VMEM((CHC, D_MODEL), jnp.float32),
            pltpu.VMEM((CHC, D_MODEL), jnp.float32),
            pltpu.VMEM((CHC, D_MODEL), jnp.float32),
            pltpu.SemaphoreType.DMA,
            pltpu.SemaphoreType.DMA,
        ],
    )(_combine_body)
    out = combine(yg, pose_c, poso_c)
    return out.reshape(b, s, d)


# speedup vs baseline: 2.0493x; 2.0493x over previous
import functools

import jax
import jax.numpy as jnp
from jax import lax
from jax.experimental import pallas as pl
from jax.experimental.pallas import tpu as pltpu
from jax.experimental.pallas import tpu_sc as plsc

D_MODEL = 1024
D_FF = 2048
N_EXP = 8
N_TOK = 4096
N_PAIR = 2 * N_TOK
TD = 256
NB = N_PAIR // TD + N_EXP
NPAD = NB * TD
NW = 32
TPW = N_TOK // NW
CHD = 32
CHC = 16
WGL = 128

_HI = jax.lax.Precision.HIGHEST


def _router_body(x_ref, wr_ref, i1_ref, i2_ref, wa_ref, wb_ref):
    xb = x_ref[...]
    logits = lax.dot_general(xb, wr_ref[...], (((1,), (1,)), ((), ())),
                             preferred_element_type=jnp.float32)
    p = jax.nn.softmax(logits, axis=-1)
    i1 = jnp.argmax(p, axis=-1)
    v1 = jnp.max(p, axis=-1)
    lane = lax.broadcasted_iota(jnp.int32, p.shape, 1)
    pm = jnp.where(lane == i1[:, None], -1.0, p)
    i2 = jnp.argmax(pm, axis=-1)
    v2 = jnp.max(pm, axis=-1)
    denom = v1 + v2
    i1_ref[...] = i1.astype(jnp.int32)
    i2_ref[...] = i2.astype(jnp.int32)
    wa_ref[...] = v1 / denom
    wb_ref[...] = v2 / denom


def _book_body(pe_ref, pos_ref, beo_ref):
    pe = pe_ref[...]
    r128 = lax.broadcasted_iota(jnp.int32, (128, 128), 0)
    c128 = lax.broadcasted_iota(jnp.int32, (128, 128), 1)
    U = (r128 <= c128).astype(jnp.float32)
    r64 = lax.broadcasted_iota(jnp.int32, (64, 64), 0)
    c64 = lax.broadcasted_iota(jnp.int32, (64, 64), 1)
    L = (c64 < r64).astype(jnp.float32)

    pos_acc = jnp.zeros((64, 128), jnp.float32)
    be_acc = jnp.zeros((8, 128), jnp.int32)
    bidx = lax.broadcasted_iota(jnp.int32, (8, 128), 1)
    offs = jnp.int32(0)
    for e in range(N_EXP):
        m = (pe == e).astype(jnp.float32)
        cs = lax.dot_general(m, U, (((1,), (0,)), ((), ())),
                             precision=_HI, preferred_element_type=jnp.float32)
        rt = cs[:, 127:128]
        O = lax.dot_general(L, rt, (((1,), (0,)), ((), ())),
                            precision=_HI, preferred_element_type=jnp.float32)
        cnt = jnp.sum(rt).astype(jnp.int32)
        offs_f = offs.astype(jnp.float32)
        pos_acc = pos_acc + m * (offs_f + O + cs - 1.0)
        be_acc = be_acc + (bidx >= offs // TD).astype(jnp.int32)
        pc = ((cnt + TD - 1) // TD) * TD
        offs = offs + pc
    pos_ref[...] = pos_acc.astype(jnp.int32)
    beo_ref[...] = be_acc - 1


def _dispatch_body(x_hbm, pose_hbm, poso_hbm, w0_hbm, w1_hbm,
                   xg_hbm, wg_hbm,
                   pose_v, poso_v, w0_v, w1_v,
                   rows_a, rows_b, wg0a, wg0b, wg1a, wg1b, semr, semw):
    wid = lax.axis_index("s") * 2 + lax.axis_index("c")
    pltpu.sync_copy(pose_hbm.at[wid], pose_v)
    pltpu.sync_copy(poso_hbm.at[wid], poso_v)
    pltpu.sync_copy(w0_hbm.at[pl.ds(wid * TPW, TPW)], w0_v)
    pltpu.sync_copy(w1_hbm.at[pl.ds(wid * TPW, TPW)], w1_v)
    nch = TPW // CHD
    rows = (rows_a, rows_b)
    wg0 = (wg0a, wg0b)
    wg1 = (wg1a, wg1b)
    rd = [None] * nch
    wr = [None] * nch
    rd[0] = pltpu.async_copy(x_hbm.at[pl.ds(wid * TPW, CHD)], rows_a, semr)
    for c in range(nch):
        if c >= 1:
            for d in wr[c - 1]:
                d.wait()
        if c + 1 < nch:
            rd[c + 1] = pltpu.async_copy(
                x_hbm.at[pl.ds(wid * TPW + (c + 1) * CHD, CHD)],
                rows[(c + 1) % 2], semr)
        b = c % 2
        for i in range(CHD):
            v0 = w0_v[pl.ds(c * CHD + (i // 16) * 16, 16)]
            v1 = w1_v[pl.ds(c * CHD + (i // 16) * 16, 16)]
            f0 = jnp.full((16,), v0[i % 16], jnp.float32)
            f1 = jnp.full((16,), v1[i % 16], jnp.float32)
            for j in range(WGL // 16):
                wg0[b][i, pl.ds(j * 16, 16)] = f0
                wg1[b][i, pl.ds(j * 16, 16)] = f1
        rd[c].wait()
        wr[c] = [
            pltpu.async_copy(rows[b], xg_hbm.at[pose_v.at[c]], semw),
            pltpu.async_copy(rows[b], xg_hbm.at[poso_v.at[c]], semw),
            pltpu.async_copy(wg0[b], wg_hbm.at[pose_v.at[c]], semw),
            pltpu.async_copy(wg1[b], wg_hbm.at[poso_v.at[c]], semw),
        ]
    for d in wr[nch - 1]:
        d.wait()


def _ffn_body(be_ref, xg_ref, w1_ref, w2_ref, wg_ref, yg_ref):
    xb = xg_ref[...].astype(jnp.bfloat16)
    h = lax.dot_general(xb, w1_ref[0], (((1,), (1,)), ((), ())),
                        preferred_element_type=jnp.float32)
    h = (h * jax.nn.sigmoid(h)).astype(jnp.bfloat16)
    y = lax.dot_general(h, w2_ref[0], (((1,), (1,)), ((), ())),
                        preferred_element_type=jnp.float32)
    yg_ref[...] = y * wg_ref[...][:, 0:1]


def _combine_body(yg_hbm, pose_hbm, poso_hbm, out_hbm,
                  pose_v, poso_v, ye_a, ye_b, yo_a, yo_b, o_a, o_b, semr, semw):
    wid = lax.axis_index("s") * 2 + lax.axis_index("c")
    pltpu.sync_copy(pose_hbm.at[wid], pose_v)
    pltpu.sync_copy(poso_hbm.at[wid], poso_v)
    nch = TPW // CHC
    ye = (ye_a, ye_b)
    yo = (yo_a, yo_b)
    ov = (o_a, o_b)
    gd = [None] * nch
    ww = [None] * nch
    gd[0] = [pltpu.async_copy(yg_hbm.at[pose_v.at[0]], ye_a, semr),
             pltpu.async_copy(yg_hbm.at[poso_v.at[0]], yo_a, semr)]
    for c in range(nch):
        b = c % 2
        if c + 1 < nch:
            nb = (c + 1) % 2
            gd[c + 1] = [
                pltpu.async_copy(yg_hbm.at[pose_v.at[c + 1]], ye[nb], semr),
                pltpu.async_copy(yg_hbm.at[poso_v.at[c + 1]], yo[nb], semr)]
        for d in gd[c]:
            d.wait()
        if c >= 2:
            ww[c - 2].wait()

        def row(i, _, b=b):
            for j in range(D_MODEL // 16):
                sl = pl.ds(j * 16, 16)
                ov[b][i, sl] = ye[b][i, sl] + yo[b][i, sl]
            return 0

        lax.fori_loop(0, CHC, row, 0)
        ww[c] = pltpu.async_copy(
            ov[b], out_hbm.at[pl.ds(wid * TPW + c * CHC, CHC)], semw)
    ww[nch - 2].wait()
    ww[nch - 1].wait()


@jax.jit
def kernel(x, Wr, W1, W2):
    b, s, d = x.shape
    x2 = x.reshape(N_TOK, d)
    w1b = W1.astype(jnp.bfloat16)
    w2b = W2.astype(jnp.bfloat16)

    i1, i2, wa, wb = pl.pallas_call(
        _router_body,
        out_shape=[
            jax.ShapeDtypeStruct((N_TOK,), jnp.int32),
            jax.ShapeDtypeStruct((N_TOK,), jnp.int32),
            jax.ShapeDtypeStruct((N_TOK,), jnp.float32),
            jax.ShapeDtypeStruct((N_TOK,), jnp.float32),
        ],
    )(x2, Wr)

    pair_e = jnp.stack([i1, i2], axis=1).reshape(64, 128)
    pos, beo = pl.pallas_call(
        _book_body,
        out_shape=[
            jax.ShapeDtypeStruct((64, 128), jnp.int32),
            jax.ShapeDtypeStruct((8, 128), jnp.int32),
        ],
    )(pair_e)

    pos2 = pos.reshape(N_TOK, 2)
    pose_d = pos2[:, 0].reshape(NW, TPW // CHD, CHD)
    poso_d = pos2[:, 1].reshape(NW, TPW // CHD, CHD)
    pose_c = pos2[:, 0].reshape(NW, TPW // CHC, CHC)
    poso_c = pos2[:, 1].reshape(NW, TPW // CHC, CHC)
    be = beo[0, :NB]

    mesh = plsc.VectorSubcoreMesh(core_axis_name="c", subcore_axis_name="s")
    dispatch = functools.partial(
        pl.kernel,
        mesh=mesh,
        out_type=[
            jax.ShapeDtypeStruct((NPAD, D_MODEL), jnp.float32),
            jax.ShapeDtypeStruct((NPAD, WGL), jnp.float32),
        ],
        scratch_types=[
            pltpu.VMEM((TPW // CHD, CHD), jnp.int32),
            pltpu.VMEM((TPW // CHD, CHD), jnp.int32),
            pltpu.VMEM((TPW,), jnp.float32),
            pltpu.VMEM((TPW,), jnp.float32),
            pltpu.VMEM((CHD, D_MODEL), jnp.float32),
            pltpu.VMEM((CHD, D_MODEL), jnp.float32),
            pltpu.VMEM((CHD, WGL), jnp.float32),
            pltpu.VMEM((CHD, WGL), jnp.float32),
            pltpu.VMEM((CHD, WGL), jnp.float32),
            pltpu.VMEM((CHD, WGL), jnp.float32),
            pltpu.SemaphoreType.DMA,
            pltpu.SemaphoreType.DMA,
        ],
    )(_dispatch_body)
    xg, wg = dispatch(x2, pose_d, poso_d, wa, wb)

    grid_spec = pltpu.PrefetchScalarGridSpec(
        num_scalar_prefetch=1,
        grid=(NB,),
        in_specs=[
            pl.BlockSpec((TD, D_MODEL), lambda bb, be_r: (bb, 0)),
            pl.BlockSpec((1, D_FF, D_MODEL), lambda bb, be_r: (be_r[bb], 0, 0)),
            pl.BlockSpec((1, D_MODEL, D_FF), lambda bb, be_r: (be_r[bb], 0, 0)),
            pl.BlockSpec((TD, WGL), lambda bb, be_r: (bb, 0)),
        ],
        out_specs=pl.BlockSpec((TD, D_MODEL), lambda bb, be_r: (bb, 0)),
    )
    yg = pl.pallas_call(
        _ffn_body,
        grid_spec=grid_spec,
        out_shape=jax.ShapeDtypeStruct((NPAD, D_MODEL), jnp.float32),
    )(be, xg, w1b, w2b, wg)

    combine = functools.partial(
        pl.kernel,
        mesh=mesh,
        out_type=jax.ShapeDtypeStruct((N_TOK, D_MODEL), jnp.float32),
        scratch_types=[
            pltpu.VMEM((TPW // CHC, CHC), jnp.int32),
            pltpu.VMEM((TPW // CHC, CHC), jnp.int32),
            pltpu.VMEM((CHC, D_MODEL), jnp.float32),
            pltpu.VMEM((CHC, D_MODEL), jnp.float32),
            pltpu.VMEM((CHC, D_MODEL), jnp.float32),
            pltpu.VMEM((CHC, D_MODEL), jnp.float32),
            pltpu.VMEM((CHC, D_MODEL), jnp.float32),
            pltpu.VMEM((CHC, D_MODEL), jnp.float32),
            pltpu.SemaphoreType.DMA,
            pltpu.SemaphoreType.DMA,
        ],
    )(_combine_body)
    out = combine(yg, pose_c, poso_c)
    return out.reshape(b, s, d)
